# Initial kernel scaffold; baseline (speedup 1.0000x reference)
import jax
import jax.numpy as jnp
from jax.experimental import pallas as pl


def kernel(q_len, k_len, bidirectional, table):
    raise NotImplementedError("write your pallas kernel here")


# speedup vs baseline: 54.5917x; 54.5917x over previous
import math

import jax
import jax.numpy as jnp
from jax.experimental import pallas as pl
from jax.experimental.pallas import tpu as pltpu

_NUM_BUCKETS = 32
_MAX_DISTANCE = 128
_NUM_HEADS = 16
_SEQ = 2048
_LINE = 2 * _SEQ
_BQ = 16


def _bias_body(table_ref, out_ref, line_ref):
    qi = pl.program_id(0)

    @pl.when(qi == 0)
    def _build_line():
        j = jax.lax.broadcasted_iota(jnp.int32, (_NUM_BUCKETS, _LINE), 1)
        d = j - (_SEQ - 1)
        n = -d
        half = _NUM_BUCKETS // 2
        sign = (n < 0).astype(jnp.int32)
        n = jnp.abs(n)
        max_exact = half // 2
        is_small = n < max_exact
        val_if_large = max_exact + (
            jnp.log(n.astype(jnp.float32) / max_exact + 1e-06)
            / math.log(_MAX_DISTANCE / max_exact)
            * (half - max_exact)
        ).astype(jnp.int32)
        val_if_large = jnp.minimum(val_if_large, half - 1)
        bucket = jnp.where(is_small, n, val_if_large) + sign * half
        b_iota = jax.lax.broadcasted_iota(jnp.int32, (_NUM_BUCKETS, _LINE), 0)
        onehot = (bucket == b_iota).astype(jnp.float32)
        line_ref[...] = jax.lax.dot_general(
            table_ref[...], onehot,
            dimension_numbers=(((0,), (0,)), ((), ())),
            preferred_element_type=jnp.float32)

    shift = qi * _BQ + (_SEQ + 1)
    for h in range(_NUM_HEADS):
        row = line_ref[h, :]
        x = jax.lax.broadcast_in_dim(row, (_BQ, _LINE), (1,))
        rolled = pltpu.roll(pltpu.roll(x, shift, 1), 0, 1,
                            stride=1, stride_axis=0)
        out_ref[0, h, :, :] = rolled[:, :_SEQ]


def kernel(q_len, k_len, bidirectional, table):
    del q_len, k_len, bidirectional
    return pl.pallas_call(
        _bias_body,
        grid=(_SEQ // _BQ,),
        in_specs=[pl.BlockSpec((_NUM_BUCKETS, _NUM_HEADS), lambda i: (0, 0))],
        out_specs=pl.BlockSpec(
            (1, _NUM_HEADS, _BQ, _SEQ), lambda i: (0, 0, i, 0)),
        out_shape=jax.ShapeDtypeStruct(
            (1, _NUM_HEADS, _SEQ, _SEQ), jnp.float32),
        scratch_shapes=[pltpu.VMEM((_NUM_HEADS, _LINE), jnp.float32)],
        compiler_params=pltpu.CompilerParams(
            dimension_semantics=("arbitrary",)),
    )(table)


# speedup vs baseline: 176.7432x; 3.2375x over previous
import math

import jax
import jax.numpy as jnp
from jax.experimental import pallas as pl
from jax.experimental.pallas import tpu as pltpu

_NUM_BUCKETS = 32
_MAX_DISTANCE = 128
_NUM_HEADS = 16
_SEQ = 2048
_LINE = 2 * _SEQ
_BQ = 128
_W = _SEQ + _BQ


def _bias_body(tablet_ref, out_ref, line_ref):
    qi = pl.program_id(0)

    @pl.when(qi == 0)
    def _build_line():
        j = jax.lax.broadcasted_iota(jnp.int32, (_NUM_HEADS, _LINE), 1)
        d = j - (_SEQ - 1)
        n = -d
        half = _NUM_BUCKETS // 2
        sign = (n < 0).astype(jnp.int32)
        n = jnp.abs(n)
        max_exact = half // 2
        is_small = n < max_exact
        val_if_large = max_exact + (
            jnp.log(n.astype(jnp.float32) / max_exact + 1e-06)
            / math.log(_MAX_DISTANCE / max_exact)
            * (half - max_exact)
        ).astype(jnp.int32)
        val_if_large = jnp.minimum(val_if_large, half - 1)
        bucket = jnp.where(is_small, n, val_if_large) + sign * half
        acc = jnp.zeros((_NUM_HEADS, _LINE), jnp.float32)
        for b in range(_NUM_BUCKETS):
            col = jax.lax.broadcast_in_dim(
                tablet_ref[:, b:b + 1], (_NUM_HEADS, _LINE), (0, 1))
            acc = jnp.where(bucket == b, col, acc)
        line_ref[...] = acc

    base = pl.multiple_of((pl.num_programs(0) - 1 - qi) * _BQ, _BQ)
    for h in range(_NUM_HEADS):
        w = line_ref[h, pl.ds(base, _W)]
        x = jax.lax.broadcast_in_dim(w, (_BQ, _W), (1,))
        rolled = pltpu.roll(x, _W - (_BQ - 1), 1, stride=1, stride_axis=0)
        out_ref[0, h, :, :] = rolled[:, :_SEQ]


def kernel(q_len, k_len, bidirectional, table):
    del q_len, k_len, bidirectional
    return pl.pallas_call(
        _bias_body,
        grid=(_SEQ // _BQ,),
        in_specs=[pl.BlockSpec((_NUM_HEADS, _NUM_BUCKETS), lambda i: (0, 0))],
        out_specs=pl.BlockSpec(
            (1, _NUM_HEADS, _BQ, _SEQ), lambda i: (0, 0, i, 0)),
        out_shape=jax.ShapeDtypeStruct(
            (1, _NUM_HEADS, _SEQ, _SEQ), jnp.float32),
        scratch_shapes=[pltpu.VMEM((_NUM_HEADS, _LINE), jnp.float32)],
        compiler_params=pltpu.CompilerParams(
            dimension_semantics=("arbitrary",)),
    )(table.T)
